# Initial kernel scaffold; baseline (speedup 1.0000x reference)
import jax
import jax.numpy as jnp
from jax.experimental import pallas as pl


def kernel(inputs, table):
    raise NotImplementedError("write your pallas kernel here")


# speedup vs baseline: 1.5602x; 1.5602x over previous
import functools

import jax
import jax.numpy as jnp
from jax import lax
from jax.experimental import pallas as pl
from jax.experimental.pallas import tpu as pltpu
from jax.experimental.pallas import tpu_sc as plsc

BATCH = 16384
FIELDS = 26
EMBED_DIM = 32
N = BATCH * FIELDS
NC, NS = 2, 16
NW = NC * NS
RPW = N // NW
CHUNK = 128
GROUP = 8
STEP_ROWS = CHUNK * GROUP
STEPS = RPW // STEP_ROWS
CPW = RPW // CHUNK


def _sc_gather(idx2d, table):
    mesh = plsc.VectorSubcoreMesh(
        core_axis_name="c", subcore_axis_name="s",
        num_cores=NC, num_subcores=NS)

    @functools.partial(
        pl.kernel,
        out_type=jax.ShapeDtypeStruct((N, EMBED_DIM), jnp.float32),
        mesh=mesh,
        scratch_types=[
            pltpu.VMEM((CPW, CHUNK), jnp.int32),
            pltpu.VMEM((STEP_ROWS, EMBED_DIM), jnp.float32),
            pltpu.SemaphoreType.DMA,
        ],
        compiler_params=pltpu.CompilerParams(use_tc_tiling_on_sc=False),
    )
    def k(idx_hbm, table_hbm, out_hbm, idx_v, rows_v, sem):
        wid = lax.axis_index("s") * NC + lax.axis_index("c")
        pltpu.sync_copy(idx_hbm.at[pl.ds(wid * CPW, CPW)], idx_v)

        @pl.loop(0, STEPS)
        def step(t):
            base = t * GROUP
            copies = [
                pltpu.async_copy(
                    table_hbm.at[idx_v.at[base + j]],
                    rows_v.at[pl.ds(j * CHUNK, CHUNK)],
                    sem)
                for j in range(GROUP)
            ]
            for cpy in copies:
                cpy.wait()
            out0 = wid * RPW + t * STEP_ROWS
            pltpu.sync_copy(rows_v, out_hbm.at[pl.ds(out0, STEP_ROWS)])

    return k(idx2d, table)


def kernel(inputs, table):
    idx = inputs.astype(jnp.int32).reshape(N // CHUNK, CHUNK)
    out = _sc_gather(idx, table)
    return out.reshape(BATCH, FIELDS, EMBED_DIM)


# speedup vs baseline: 1.5632x; 1.0019x over previous
import functools

import jax
import jax.numpy as jnp
from jax import lax
from jax.experimental import pallas as pl
from jax.experimental.pallas import tpu as pltpu
from jax.experimental.pallas import tpu_sc as plsc

BATCH = 16384
FIELDS = 26
EMBED_DIM = 32
N = BATCH * FIELDS
NC, NS = 2, 16
NW = NC * NS
RPW = N // NW
CHUNK = 128
GROUP = 4
STEP_ROWS = CHUNK * GROUP
STEPS = RPW // STEP_ROWS
CPW = RPW // CHUNK


def _sc_gather(idx2d, table):
    mesh = plsc.VectorSubcoreMesh(
        core_axis_name="c", subcore_axis_name="s",
        num_cores=NC, num_subcores=NS)

    @functools.partial(
        pl.kernel,
        out_type=jax.ShapeDtypeStruct((N, EMBED_DIM), jnp.float32),
        mesh=mesh,
        scratch_types=[
            pltpu.VMEM((CPW, CHUNK), jnp.int32),
            pltpu.VMEM((STEP_ROWS, EMBED_DIM), jnp.float32),
            pltpu.VMEM((STEP_ROWS, EMBED_DIM), jnp.float32),
            pltpu.SemaphoreType.DMA,
            pltpu.SemaphoreType.DMA,
            pltpu.SemaphoreType.DMA,
            pltpu.SemaphoreType.DMA,
        ],
        compiler_params=pltpu.CompilerParams(use_tc_tiling_on_sc=False),
    )
    def k(idx_hbm, table_hbm, out_hbm, idx_v, rows0, rows1, sg0, sg1, sw0, sw1):
        wid = lax.axis_index("s") * NC + lax.axis_index("c")
        pltpu.sync_copy(idx_hbm.at[pl.ds(wid * CPW, CPW)], idx_v)
        rows = (rows0, rows1)
        sg = (sg0, sg1)
        sw = (sw0, sw1)

        def fire_g(b, t):
            for j in range(GROUP):
                pltpu.async_copy(
                    table_hbm.at[idx_v.at[t * GROUP + j]],
                    rows[b].at[pl.ds(j * CHUNK, CHUNK)],
                    sg[b])

        def drain_g(b):
            for j in range(GROUP):
                pltpu.make_async_copy(
                    table_hbm.at[idx_v.at[j]],
                    rows[b].at[pl.ds(j * CHUNK, CHUNK)],
                    sg[b]).wait()

        def fire_w(b, t):
            pltpu.async_copy(
                rows[b], out_hbm.at[pl.ds(wid * RPW + t * STEP_ROWS,
                                          STEP_ROWS)], sw[b])

        def wait_w(b):
            pltpu.make_async_copy(
                rows[b], out_hbm.at[pl.ds(wid * RPW, STEP_ROWS)],
                sw[b]).wait()

        fire_g(0, 0)
        fire_g(1, 1)

        @pl.loop(0, STEPS // 2 - 1)
        def body(i):
            t = i * 2
            drain_g(0)
            fire_w(0, t)
            drain_g(1)
            fire_w(1, t + 1)
            wait_w(0)
            fire_g(0, t + 2)
            wait_w(1)
            fire_g(1, t + 3)

        drain_g(0)
        fire_w(0, STEPS - 2)
        drain_g(1)
        fire_w(1, STEPS - 1)
        wait_w(0)
        wait_w(1)

    return k(idx2d, table)


def kernel(inputs, table):
    idx = inputs.astype(jnp.int32).reshape(N // CHUNK, CHUNK)
    out = _sc_gather(idx, table)
    return out.reshape(BATCH, FIELDS, EMBED_DIM)
